# baseline (speedup 1.0000x reference)
import functools

import jax
import jax.numpy as jnp
from jax import lax
from jax.experimental import pallas as pl
from jax.experimental.pallas import tpu as pltpu
from jax.experimental.pallas import tpu_sc as plsc

_B = 16384
_D = 64
_NE = 1000000
_NR = 1000
_NC = 2
_NS = 16
_NW = _NC * _NS
_T = _B // _NW
_C = 128
_G = _C // 16


def _make_sc_kernel():
    mesh = plsc.VectorSubcoreMesh(core_axis_name="c", subcore_axis_name="s")

    @functools.partial(
        pl.kernel,
        mesh=mesh,
        compiler_params=pltpu.CompilerParams(needs_layout_passes=False),
        out_type=jax.ShapeDtypeStruct((_NW, 16), jnp.float32),
        scratch_types=[
            pltpu.VMEM((_C,), jnp.int32),
            pltpu.VMEM((_C,), jnp.int32),
            pltpu.VMEM((_C,), jnp.int32),
            pltpu.VMEM((_C,), jnp.int32),
            pltpu.VMEM((_C,), jnp.int32),
            pltpu.VMEM((_C,), jnp.int32),
            pltpu.VMEM((_C,), jnp.int32),
            pltpu.VMEM((_C,), jnp.int32),
            pltpu.VMEM((_C,), jnp.int32),
            pltpu.VMEM((_C,), jnp.int32),
            pltpu.VMEM((_C, 2 * _D), jnp.float32),
            pltpu.VMEM((_C, 2 * _D), jnp.float32),
            pltpu.VMEM((_C, 2 * _D), jnp.float32),
            pltpu.VMEM((_C, 2 * _D), jnp.float32),
            pltpu.VMEM((_C, 2 * _D), jnp.float32),
            pltpu.VMEM((16,), jnp.float32),
            pltpu.SemaphoreType.DMA,
        ],
    )
    def sc_kernel(ph, pr, pt, nh, nt, node2, edge2, out,
                  ph_i, pr_i, pt_i, nh_i, nt_i,
                  ph_p, pr_p, pt_p, nh_p, nt_p,
                  hb, rb, tb, nhb, ntb, accv, sem):
        wid = lax.axis_index("s") * _NC + lax.axis_index("c")
        base = wid * _T
        lane = lax.iota(jnp.int32, 16)

        worker_acc = jnp.zeros((16,), jnp.float32)
        for c in range(_T // _C):
            off = base + c * _C
            pltpu.sync_copy(ph.at[pl.ds(off, _C)], ph_i)
            pltpu.sync_copy(pr.at[pl.ds(off, _C)], pr_i)
            pltpu.sync_copy(pt.at[pl.ds(off, _C)], pt_i)
            pltpu.sync_copy(nh.at[pl.ds(off, _C)], nh_i)
            pltpu.sync_copy(nt.at[pl.ds(off, _C)], nt_i)
            for q in range(_C // 16):
                s = pl.ds(q * 16, 16)
                ph_p[s] = ph_i[s] >> 1
                pr_p[s] = pr_i[s] >> 1
                pt_p[s] = pt_i[s] >> 1
                nh_p[s] = nh_i[s] >> 1
                nt_p[s] = nt_i[s] >> 1
            copies = [
                pltpu.async_copy(node2.at[ph_p], hb, sem),
                pltpu.async_copy(edge2.at[pr_p], rb, sem),
                pltpu.async_copy(node2.at[pt_p], tb, sem),
                pltpu.async_copy(node2.at[nh_p], nhb, sem),
                pltpu.async_copy(node2.at[nt_p], ntb, sem),
            ]
            for cp in copies:
                cp.wait()

            def g_body(g, wacc):
                rows = g * 16 + lane
                hc = (plsc.load_gather(ph_i, [rows]) & 1) * _D
                rc = (plsc.load_gather(pr_i, [rows]) & 1) * _D
                tc = (plsc.load_gather(pt_i, [rows]) & 1) * _D
                nhc = (plsc.load_gather(nh_i, [rows]) & 1) * _D
                ntc = (plsc.load_gather(nt_i, [rows]) & 1) * _D

                def j_body(j, carry):
                    ap, an = carry
                    hv = plsc.load_gather(hb, [rows, hc + j])
                    rv = plsc.load_gather(rb, [rows, rc + j])
                    tv = plsc.load_gather(tb, [rows, tc + j])
                    nhv = plsc.load_gather(nhb, [rows, nhc + j])
                    ntv = plsc.load_gather(ntb, [rows, ntc + j])
                    return (ap + jnp.abs(hv + rv - tv),
                            an + jnp.abs(nhv + rv - ntv))

                zeros = jnp.zeros((16,), jnp.float32)
                ap, an = lax.fori_loop(0, _D, j_body, (zeros, zeros))
                return wacc + jnp.maximum(an - ap, 0.0)

            worker_acc = lax.fori_loop(0, _G, g_body, worker_acc)

        accv[...] = worker_acc * (1.0 / _B)
        pltpu.sync_copy(accv, out.at[wid])

    return sc_kernel


_sc_kernel = _make_sc_kernel()


def kernel(pos_h, pos_r, pos_t, neg_h, neg_t, node_em, edge_em):
    node2 = node_em.reshape(_NE // 2, 2 * _D)
    edge2 = edge_em.reshape(_NR // 2, 2 * _D)
    partials = _sc_kernel(pos_h, pos_r, pos_t, neg_h, neg_t, node2, edge2)
    return jnp.sum(partials)
